# baseline (speedup 1.0000x reference)
import jax
import jax.numpy as jnp
from jax.experimental import pallas as pl

N_UAV = 512
N_TGT = 256
H = 64
HEADS = 4

_F32 = jnp.float32

_P_SPECS = (('b1', 256), ('g1', 256), ('be1', 256), ('b3', 256), ('g3', 256),
            ('be3', 256), ('b2', 64), ('g2', 64), ('be2', 64), ('b4', 64),
            ('g4', 64), ('be4', 64), ('bt', 128), ('bf1', 64), ('bf2', 32))
_P_OFF = {}
_off = 0
for _n, _w in _P_SPECS:
    _P_OFF[_n] = (_off, _w)
    _off += _w
_P_TOT = _off

_A_OFF = {'a1s': (0, 4), 'a1d': (4, 4), 'a2s': (8, 1), 'a2d': (9, 1),
          'a3s': (10, 4), 'a3d': (14, 4), 'a4s': (18, 1), 'a4d': (19, 1)}
_DN_NN = (((1,), (0,)), ((), ()))
_DN_TN = (((0,), (0,)), ((), ()))
_DN_NT = (((1,), (1,)), ((), ()))


def _split(x):
    xh = x.astype(jnp.bfloat16)
    xl = (x - xh.astype(_F32)).astype(jnp.bfloat16)
    return xh, xl


def _d(x, y, dn):
    return jax.lax.dot_general(x, y, dn, preferred_element_type=_F32)


def _dot3(ap, bp, dn):
    ah, al = ap
    bh, bl = bp
    return _d(ah, bh, dn) + _d(al, bh, dn) + _d(ah, bl, dn)


def _row(v, h):
    return jax.lax.dot_general(v, h, _DN_NT, preferred_element_type=_F32,
                               precision=jax.lax.Precision.HIGHEST)


def _col(h, v):
    return jax.lax.dot_general(h, v, _DN_NT, preferred_element_type=_F32,
                               precision=jax.lax.Precision.HIGHEST)


def _att_dm(hs, hsp, hd, a_s, a_d, maskf):
    al = _row(a_s, hs)
    ar = _col(hd, a_d)
    e = al + ar
    e = jnp.maximum(e, 0.2 * e)
    emax = jnp.max(e, axis=1, keepdims=True)
    ee = jnp.exp(e - emax) * maskf
    den = jnp.sum(ee, axis=1, keepdims=True)
    out = _dot3(_split(ee), hsp, _DN_NN)
    return out / (den + 1e-16)


def _att_sm(hs, hsp, hd, a_s, a_d, maskf):
    al = _col(hs, a_s)
    ar = _row(a_d, hd)
    e = al + ar
    e = jnp.maximum(e, 0.2 * e)
    emax = jnp.max(e, axis=0, keepdims=True)
    ee = jnp.exp(e - emax) * maskf
    den = jnp.sum(ee, axis=0, keepdims=True)
    alpha = ee / (den + 1e-16)
    return _dot3(_split(alpha), hsp, _DN_TN)


def _bn(x, g, b, n):
    mu = jnp.sum(x, axis=0, keepdims=True) / n
    var = jnp.sum((x - mu) ** 2, axis=0, keepdims=True) / n
    return (x - mu) * jax.lax.rsqrt(var + 1e-5) * g + b


def _elu(x):
    return jnp.where(x > 0, x, jnp.exp(x) - 1.0)


def _fwd(uf_ref, tf_ref, ua_ref, ta_ref, W1_ref, W2_ref, W3_ref, W4_ref,
         Wt_ref, Wf1_ref, Wf2_ref, av_ref, pv_ref, out_ref):
    uf = uf_ref[:]
    tf = tf_ref[:]
    av = av_ref[:]
    pv = pv_ref[:]

    def P(name):
        off, w = _P_OFF[name]
        return pv[0:1, off:off + w]

    def A(name):
        off, rows = _A_OFF[name]
        return av[off:off + rows]

    rows = jax.lax.broadcasted_iota(jnp.int32, (N_UAV, N_UAV), 0)
    cols = jax.lax.broadcasted_iota(jnp.int32, (N_UAV, N_UAV), 1)
    m1 = jnp.where(jnp.logical_or(rows == cols, ua_ref[:] != 0.0), 1.0, 0.0)

    h1 = jnp.dot(uf, W1_ref[:], preferred_element_type=_F32)
    a1s = A('a1s')
    a1d = A('a1d')
    outs = []
    for k in range(HEADS):
        hk = h1[:, k * H:(k + 1) * H]
        outs.append(_att_sm(hk, _split(hk), hk, a1s[k:k + 1],
                            a1d[k:k + 1], m1))
    x1 = jnp.concatenate(outs, axis=1) + P('b1')
    x1 = _elu(_bn(x1, P('g1'), P('be1'), float(N_UAV)))

    h2 = jnp.dot(x1, W2_ref[:], preferred_element_type=_F32)
    x2 = _att_sm(h2, _split(h2), h2, A('a2s'), A('a2d'), m1) + P('b2')
    uav_h = _elu(_bn(x2, P('g2'), P('be2'), float(N_UAV)))

    mt = jnp.where(ta_ref[:] != 0.0, 1.0, 0.0)

    tproc = jnp.dot(tf, Wt_ref[:], preferred_element_type=_F32) + P('bt')
    h3u = jnp.dot(uf, W3_ref[:], preferred_element_type=_F32)
    h3t = jnp.dot(tproc, W3_ref[:], preferred_element_type=_F32)
    a3s = A('a3s')
    a3d = A('a3d')
    outs_u, outs_t = [], []
    for k in range(HEADS):
        hu = h3u[:, k * H:(k + 1) * H]
        ht = h3t[:, k * H:(k + 1) * H]
        hup = _split(hu)
        htp = _split(ht)
        outs_u.append(_att_dm(ht, htp, hu, a3s[k:k + 1], a3d[k:k + 1], mt))
        outs_t.append(_att_sm(hu, hup, ht, a3s[k:k + 1], a3d[k:k + 1], mt))
    y = jnp.concatenate(
        [jnp.concatenate(outs_u, axis=1), jnp.concatenate(outs_t, axis=1)],
        axis=0) + P('b3')
    y = _elu(_bn(y, P('g3'), P('be3'), float(N_UAV + N_TGT)))

    h4 = jnp.dot(y, W4_ref[:], preferred_element_type=_F32)
    h4u = h4[:N_UAV]
    h4t = h4[N_UAV:]
    h4up = _split(h4u)
    h4tp = _split(h4t)
    y2u = _att_dm(h4t, h4tp, h4u, A('a4s'), A('a4d'), mt)
    y2t = _att_sm(h4u, h4up, h4t, A('a4s'), A('a4d'), mt)
    y2 = jnp.concatenate([y2u, y2t], axis=0) + P('b4')
    target_h = _elu(_bn(y2, P('g4'), P('be4'), float(N_UAV + N_TGT)))

    c = jnp.concatenate([uav_h, target_h[:N_UAV]], axis=1)
    hdn = jnp.dot(c, Wf1_ref[:], preferred_element_type=_F32) + P('bf1')
    hdn = jnp.maximum(hdn, 0.0)
    out_ref[:] = (jnp.dot(hdn, Wf2_ref[:], preferred_element_type=_F32)
                  + P('bf2'))


def kernel(uav_features, target_features, uav_adj, target_adj, W1, a1s, a1d,
           b1, W2, a2s, a2d, b2, W3, a3s, a3d, b3, W4, a4s, a4d, b4, g1, be1,
           g2, be2, g3, be3, g4, be4, Wt, bt, Wf1, bf1, Wf2, bf2):
    av = jnp.concatenate([a1s, a1d, a2s, a2d, a3s, a3d, a4s, a4d], axis=0)
    pv = jnp.concatenate([b1, g1, be1, b3, g3, be3, b2, g2, be2, b4, g4, be4,
                          bt, bf1, bf2]).reshape(1, -1)
    args = (uav_features, target_features, uav_adj, target_adj,
            W1, W2, W3, W4, Wt, Wf1, Wf2, av, pv)
    return pl.pallas_call(
        _fwd,
        out_shape=jax.ShapeDtypeStruct((N_UAV, H // 2), jnp.float32),
    )(*args)


# speedup vs baseline: 1.2082x; 1.2082x over previous
import jax
import jax.numpy as jnp
from jax.experimental import pallas as pl

N_UAV = 512
N_TGT = 256
H = 64
HEADS = 4

_F32 = jnp.float32
_DN_NN = (((1,), (0,)), ((), ()))
_DN_TN = (((0,), (0,)), ((), ()))
_DN_NT = (((1,), (1,)), ((), ()))


def _split(x):
    xh = x.astype(jnp.bfloat16)
    xl = (x - xh.astype(_F32)).astype(jnp.bfloat16)
    return xh, xl


def _d(x, y, dn):
    return jax.lax.dot_general(x, y, dn, preferred_element_type=_F32)


def _dot3(ap, bp, dn):
    ah, al = ap
    bh, bl = bp
    return _d(ah, bh, dn) + _d(al, bh, dn) + _d(ah, bl, dn)


def _row(v, h):
    return jax.lax.dot_general(v, h, _DN_NT, preferred_element_type=_F32,
                               precision=jax.lax.Precision.HIGHEST)


def _col(h, v):
    return jax.lax.dot_general(h, v, _DN_NT, preferred_element_type=_F32,
                               precision=jax.lax.Precision.HIGHEST)


def _att_dm(hs, hsp, hd, a_s, a_d, maskf):
    al = _row(a_s, hs)
    ar = _col(hd, a_d)
    e = al + ar
    e = jnp.maximum(e, 0.2 * e)
    emax = jnp.max(e, axis=1, keepdims=True)
    ee = jnp.exp(e - emax) * maskf
    den = jnp.sum(ee, axis=1, keepdims=True)
    out = _dot3(_split(ee), hsp, _DN_NN)
    return out / (den + 1e-16)


def _att_sm(hs, hsp, hd, a_s, a_d, maskf):
    al = _col(hs, a_s)
    ar = _row(a_d, hd)
    e = al + ar
    e = jnp.maximum(e, 0.2 * e)
    emax = jnp.max(e, axis=0, keepdims=True)
    ee = jnp.exp(e - emax) * maskf
    den = jnp.sum(ee, axis=0, keepdims=True)
    alpha = ee / (den + 1e-16)
    return _dot3(_split(alpha), hsp, _DN_TN)


def _bn(x, g, b, n):
    mu = jnp.sum(x, axis=0, keepdims=True) / n
    var = jnp.sum((x - mu) ** 2, axis=0, keepdims=True) / n
    return (x - mu) * jax.lax.rsqrt(var + 1e-5) * g + b


def _elu(x):
    return jnp.where(x > 0, x, jnp.exp(x) - 1.0)


def _fwd(uf_ref, tf_ref, ua_ref, ta_ref,
         W1_ref, a1s_ref, a1d_ref, b1_ref, W2_ref, a2s_ref, a2d_ref, b2_ref,
         W3_ref, a3s_ref, a3d_ref, b3_ref, W4_ref, a4s_ref, a4d_ref, b4_ref,
         g1_ref, be1_ref, g2_ref, be2_ref, g3_ref, be3_ref, g4_ref, be4_ref,
         Wt_ref, bt_ref, Wf1_ref, bf1_ref, Wf2_ref, bf2_ref, out_ref):
    uf = uf_ref[:]
    tf = tf_ref[:]

    rows = jax.lax.broadcasted_iota(jnp.int32, (N_UAV, N_UAV), 0)
    cols = jax.lax.broadcasted_iota(jnp.int32, (N_UAV, N_UAV), 1)
    m1 = jnp.where(jnp.logical_or(rows == cols, ua_ref[:] != 0.0), 1.0, 0.0)

    h1 = jnp.dot(uf, W1_ref[:], preferred_element_type=_F32)
    a1s = a1s_ref[:]
    a1d = a1d_ref[:]
    outs = []
    for k in range(HEADS):
        hk = h1[:, k * H:(k + 1) * H]
        outs.append(_att_sm(hk, _split(hk), hk, a1s[k:k + 1],
                            a1d[k:k + 1], m1))
    x1 = jnp.concatenate(outs, axis=1) + b1_ref[:]
    x1 = _elu(_bn(x1, g1_ref[:], be1_ref[:], float(N_UAV)))

    h2 = jnp.dot(x1, W2_ref[:], preferred_element_type=_F32)
    x2 = _att_sm(h2, _split(h2), h2, a2s_ref[:], a2d_ref[:], m1) + b2_ref[:]
    uav_h = _elu(_bn(x2, g2_ref[:], be2_ref[:], float(N_UAV)))

    mt = jnp.where(ta_ref[:] != 0.0, 1.0, 0.0)

    tproc = jnp.dot(tf, Wt_ref[:], preferred_element_type=_F32) + bt_ref[:]
    h3u = jnp.dot(uf, W3_ref[:], preferred_element_type=_F32)
    h3t = jnp.dot(tproc, W3_ref[:], preferred_element_type=_F32)
    a3s = a3s_ref[:]
    a3d = a3d_ref[:]
    outs_u, outs_t = [], []
    for k in range(HEADS):
        hu = h3u[:, k * H:(k + 1) * H]
        ht = h3t[:, k * H:(k + 1) * H]
        hup = _split(hu)
        htp = _split(ht)
        outs_u.append(_att_dm(ht, htp, hu, a3s[k:k + 1], a3d[k:k + 1], mt))
        outs_t.append(_att_sm(hu, hup, ht, a3s[k:k + 1], a3d[k:k + 1], mt))
    y = jnp.concatenate(
        [jnp.concatenate(outs_u, axis=1), jnp.concatenate(outs_t, axis=1)],
        axis=0) + b3_ref[:]
    y = _elu(_bn(y, g3_ref[:], be3_ref[:], float(N_UAV + N_TGT)))

    h4 = jnp.dot(y, W4_ref[:], preferred_element_type=_F32)
    h4u = h4[:N_UAV]
    h4t = h4[N_UAV:]
    h4up = _split(h4u)
    h4tp = _split(h4t)
    y2u = _att_dm(h4t, h4tp, h4u, a4s_ref[:], a4d_ref[:], mt)
    y2t = _att_sm(h4u, h4up, h4t, a4s_ref[:], a4d_ref[:], mt)
    y2 = jnp.concatenate([y2u, y2t], axis=0) + b4_ref[:]
    target_h = _elu(_bn(y2, g4_ref[:], be4_ref[:], float(N_UAV + N_TGT)))

    c = jnp.concatenate([uav_h, target_h[:N_UAV]], axis=1)
    hdn = jnp.dot(c, Wf1_ref[:], preferred_element_type=_F32) + bf1_ref[:]
    hdn = jnp.maximum(hdn, 0.0)
    out_ref[:] = (jnp.dot(hdn, Wf2_ref[:], preferred_element_type=_F32)
                  + bf2_ref[:])


def kernel(uav_features, target_features, uav_adj, target_adj, W1, a1s, a1d,
           b1, W2, a2s, a2d, b2, W3, a3s, a3d, b3, W4, a4s, a4d, b4, g1, be1,
           g2, be2, g3, be3, g4, be4, Wt, bt, Wf1, bf1, Wf2, bf2):
    row = lambda v: v.reshape(1, -1)
    args = (
        uav_features, target_features, uav_adj, target_adj,
        W1, a1s, a1d, row(b1), W2, a2s, a2d, row(b2),
        W3, a3s, a3d, row(b3), W4, a4s, a4d, row(b4),
        row(g1), row(be1), row(g2), row(be2),
        row(g3), row(be3), row(g4), row(be4),
        Wt, row(bt), Wf1, row(bf1), Wf2, row(bf2),
    )
    return pl.pallas_call(
        _fwd,
        out_shape=jax.ShapeDtypeStruct((N_UAV, H // 2), jnp.float32),
    )(*args)
